# baseline (speedup 1.0000x reference)
import functools

import jax
import jax.numpy as jnp
from jax import lax
from jax.experimental import pallas as pl
from jax.experimental.pallas import tpu as pltpu
from jax.experimental.pallas import tpu_sc as plsc

F32 = jnp.float32
_D = 64
_W = 128
_CH = 128
_NSC = 16


def _silu(x):
    return x * jax.nn.sigmoid(x)


def _vmesh():
    return plsc.VectorSubcoreMesh(core_axis_name="core", subcore_axis_name="subcore")


def _sc_gather(table, idx):
    K = idx.shape[0]

    @functools.partial(
        pl.kernel,
        out_type=jax.ShapeDtypeStruct((K, _W), F32),
        mesh=_vmesh())
    def k(x_hbm, i_hbm, o_hbm):
        def body(i_vmem, o_vmem):
            pltpu.sync_copy(x_hbm.at[i_vmem.at[0]], o_vmem)

        pltpu.emit_pipeline(
            body,
            grid=(K // _CH,),
            in_specs=[pl.BlockSpec((1, _CH), lambda i: (0, i))],
            out_specs=[pl.BlockSpec((_CH, _W), lambda i: (i, 0))],
            core_axis_name=("core", "subcore"),
            dimension_semantics=(pltpu.PARALLEL,),
        )(i_hbm, o_hbm)

    return k(table, idx.reshape(1, K))


_NPASS = 2


def _sc_segment_sum(msg, tgt3, n1p):
    Bb, Ep = msg.shape[0], msg.shape[1]
    nch = tgt3.shape[3]
    ept = nch * _CH
    rng = n1p // _NPASS
    rpt = rng // _NSC
    bpc = Bb // 2

    @functools.partial(
        pl.kernel,
        out_type=jax.ShapeDtypeStruct((Bb, n1p, _W), F32),
        mesh=_vmesh(),
        scratch_types=[
            pltpu.VMEM((nch, _CH), jnp.int32),
            pltpu.VMEM((ept, _W), F32),
            pltpu.VMEM((40, _W), F32),
            pltpu.VMEM_SHARED((rng + 8 * _NSC, _W), F32),
        ])
    def k(msg_hbm, tgt_hbm, o_hbm, idx_v, msg_v, zero_v, acc_sh):
        c = lax.axis_index("core")
        s = lax.axis_index("subcore")

        @pl.loop(0, 40)
        def _(r):
            for l in range(_W // 16):
                zero_v[r, pl.ds(l * 16, 16)] = jnp.zeros((16,), F32)

        for bi in range(bpc):
            b = c * bpc + bi
            pltpu.sync_copy(msg_hbm.at[b, pl.ds(s * ept, ept)], msg_v)
            for p in range(_NPASS):
                for q in range(rpt // 40):
                    pltpu.sync_copy(zero_v,
                                    acc_sh.at[pl.ds(s * rpt + q * 40, 40)])
                plsc.subcore_barrier()
                pltpu.sync_copy(tgt_hbm.at[p, b, s], idx_v)
                for j in range(nch):
                    pltpu.sync_copy(msg_v.at[pl.ds(j * _CH, _CH)],
                                    acc_sh.at[idx_v.at[j]], add=True)
                plsc.subcore_barrier()
                pltpu.sync_copy(acc_sh.at[pl.ds(s * rpt, rpt)],
                                o_hbm.at[b, pl.ds(p * rng + s * rpt, rpt)])

    return k(msg, tgt3)


def _tc_embed(nf, w1, b1, w2, b2):
    R, fin = nf.shape
    RB = R // 32

    def body(nf_ref, w1_ref, b1_ref, w2_ref, b2_ref, o_ref):
        x = jnp.dot(nf_ref[...], w1_ref[...], preferred_element_type=F32) + b1_ref[...]
        x = _silu(x)
        y = jnp.dot(x, w2_ref[...], preferred_element_type=F32) + b2_ref[...]
        o_ref[...] = jnp.concatenate([y, y], axis=1)

    return pl.pallas_call(
        body,
        grid=(R // RB,),
        in_specs=[
            pl.BlockSpec((RB, fin), lambda i: (i, 0)),
            pl.BlockSpec((fin, _D), lambda i: (0, 0)),
            pl.BlockSpec((1, _D), lambda i: (0, 0)),
            pl.BlockSpec((_D, _D), lambda i: (0, 0)),
            pl.BlockSpec((1, _D), lambda i: (0, 0)),
        ],
        out_specs=pl.BlockSpec((RB, _W), lambda i: (i, 0)),
        out_shape=jax.ShapeDtypeStruct((R, _W), F32),
    )(nf, w1, b1, w2, b2)


def _tc_msg(g, w1t, w1s, b1, w2, b2, rows):
    RB = 2048
    nb = rows // RB

    def body(t_ref, s_ref, w1t_ref, w1s_ref, b1_ref, w2_ref, b2_ref, o_ref):
        x = (jnp.dot(t_ref[...][:, :_D], w1t_ref[...],
                     preferred_element_type=F32)
             + jnp.dot(s_ref[...][:, :_D], w1s_ref[...],
                       preferred_element_type=F32)
             + b1_ref[...])
        x = _silu(x)
        m = jnp.dot(x, w2_ref[...], preferred_element_type=F32) + b2_ref[...]
        o_ref[...] = jnp.concatenate([m, jnp.zeros_like(m)], axis=1)

    wspec = lambda shp: pl.BlockSpec(shp, lambda i: (0, 0))
    return pl.pallas_call(
        body,
        grid=(nb,),
        in_specs=[
            pl.BlockSpec((RB, _W), lambda i: (i, 0)),
            pl.BlockSpec((RB, _W), lambda i: (i + nb, 0)),
            wspec((_D, _D)), wspec((_D, _D)), wspec((1, _D)),
            wspec((_D, _D)), wspec((1, _D)),
        ],
        out_specs=pl.BlockSpec((RB, _W), lambda i: (i, 0)),
        out_shape=jax.ShapeDtypeStruct((rows, _W), F32),
    )(g, g, w1t, w1s, b1, w2, b2)


def _tc_upd(h, agg, w1h, w1a, b1, w2, b2, gamma, beta):
    R = h.shape[0]
    RB = R // 32

    def body(h_ref, a_ref, w1h_ref, w1a_ref, b1_ref, w2_ref, b2_ref,
             g_ref, be_ref, o_ref):
        hh = h_ref[...][:, :_D]
        x = (jnp.dot(hh, w1h_ref[...], preferred_element_type=F32)
             + jnp.dot(a_ref[...][:, :_D], w1a_ref[...],
                       preferred_element_type=F32)
             + b1_ref[...])
        x = _silu(x)
        hn = jnp.dot(x, w2_ref[...], preferred_element_type=F32) + b2_ref[...]
        y = hh + hn
        mu = jnp.mean(y, axis=-1, keepdims=True)
        var = jnp.mean((y - mu) ** 2, axis=-1, keepdims=True)
        y = (y - mu) * lax.rsqrt(var + 1e-5) * g_ref[...] + be_ref[...]
        o_ref[...] = jnp.concatenate([y, y], axis=1)

    wspec = lambda shp: pl.BlockSpec(shp, lambda i: (0, 0))
    return pl.pallas_call(
        body,
        grid=(R // RB,),
        in_specs=[
            pl.BlockSpec((RB, _W), lambda i: (i, 0)),
            pl.BlockSpec((RB, _W), lambda i: (i, 0)),
            wspec((_D, _D)), wspec((_D, _D)), wspec((1, _D)),
            wspec((_D, _D)), wspec((1, _D)), wspec((1, _D)), wspec((1, _D)),
        ],
        out_specs=pl.BlockSpec((RB, _W), lambda i: (i, 0)),
        out_shape=jax.ShapeDtypeStruct((R, _W), F32),
    )(h, agg, w1h, w1a, b1, w2, b2, gamma, beta)


def _tc_score(g, w0s, b0, w1, b1, w2row, b2, rows):
    RB = 2048
    nb = rows // RB

    def body(g0, g1, g2, g3, w00, w01, w02, w03, b0_ref, w1_ref, b1_ref,
             w2_ref, b2_ref, o_ref):
        x = (jnp.dot(g0[...][:, :_D], w00[...], preferred_element_type=F32)
             + jnp.dot(g1[...][:, :_D], w01[...], preferred_element_type=F32)
             + jnp.dot(g2[...][:, :_D], w02[...], preferred_element_type=F32)
             + jnp.dot(g3[...][:, :_D], w03[...], preferred_element_type=F32)
             + b0_ref[...])
        x = _silu(x)
        x = _silu(jnp.dot(x, w1_ref[...], preferred_element_type=F32) + b1_ref[...])
        s = jnp.sum(x * w2_ref[...], axis=1, keepdims=True) + b2_ref[...]
        o_ref[...] = jnp.broadcast_to(s, (RB, 8))

    wspec = lambda shp: pl.BlockSpec(shp, lambda i: (0, 0))
    gspec = lambda j: pl.BlockSpec((RB, _W), lambda i, j=j: (i + j * nb, 0))
    return pl.pallas_call(
        body,
        grid=(nb,),
        in_specs=[
            gspec(0), gspec(1), gspec(2), gspec(3),
            wspec((_D, _D)), wspec((_D, _D)), wspec((_D, _D)), wspec((_D, _D)),
            wspec((1, _D)), wspec((_D, _D)), wspec((1, _D)),
            wspec((1, _D)), wspec((1, 1)),
        ],
        out_specs=pl.BlockSpec((RB, 8), lambda i: (i, 0)),
        out_shape=jax.ShapeDtypeStruct((rows, 8), F32),
    )(g, g, g, g, *w0s, b0, w1, b1, w2row, b2)


def _forward(node_features, edge_index, move_nodes, params):
    Bb, n1, fin = node_features.shape
    Ee = edge_index.shape[1]
    Mm = move_nodes.shape[1]
    nalign = _NPASS * _NSC * 8
    n1p = ((n1 + nalign - 1) // nalign) * nalign
    R = Bb * n1p
    grp = _NSC * _CH

    nf = jnp.pad(node_features, ((0, 0), (0, n1p - n1), (0, 0))).reshape(R, fin)
    ne = params["node_embed"]
    h = _tc_embed(nf, ne[0]["W"], ne[0]["b"].reshape(1, -1),
                  ne[1]["W"], ne[1]["b"].reshape(1, -1))

    offs = (jnp.arange(Bb, dtype=jnp.int32) * n1p)[:, None]
    Ep = ((Ee + grp - 1) // grp) * grp
    epad = jnp.full((Bb, Ep - Ee), n1, jnp.int32)
    tgt = jnp.concatenate([edge_index[:, :, 1], epad], axis=1)
    src = jnp.concatenate([edge_index[:, :, 0], epad], axis=1)
    idx_e = jnp.concatenate([(tgt + offs).reshape(-1), (src + offs).reshape(-1)])

    rng = n1p // _NPASS
    ept = Ep // _NSC
    dump = rng + 8 * (jnp.arange(Ep, dtype=jnp.int32) // ept)[None, :]
    tgt3 = []
    for p in range(_NPASS):
        v = tgt - p * rng
        tgt3.append(jnp.where((v >= 0) & (v < rng), v, dump))
    tgt3 = jnp.stack(tgt3).reshape(_NPASS, Bb, _NSC, Ep // grp, _CH)

    for lp in params["layers"]:
        g = _sc_gather(h, idx_e)
        w1 = lp["msg"][0]["W"]
        msg = _tc_msg(g, w1[:_D], w1[_D:], lp["msg"][0]["b"].reshape(1, -1),
                      lp["msg"][1]["W"], lp["msg"][1]["b"].reshape(1, -1),
                      Bb * Ep)
        agg = _sc_segment_sum(msg.reshape(Bb, Ep, _W), tgt3, n1p)
        wu = lp["upd"][0]["W"]
        h = _tc_upd(h, agg.reshape(R, _W), wu[:_D], wu[_D:],
                    lp["upd"][0]["b"].reshape(1, -1), lp["upd"][1]["W"],
                    lp["upd"][1]["b"].reshape(1, -1),
                    lp["gamma"].reshape(1, -1), lp["beta"].reshape(1, -1))

    Mp = ((Mm + _CH - 1) // _CH) * _CH
    mvi = jnp.concatenate(
        [move_nodes.transpose(0, 2, 1),
         jnp.full((Bb, 4, Mp - Mm), n1, jnp.int32)], axis=2)
    idx_m = (mvi + offs[:, :, None]).transpose(1, 0, 2).reshape(-1)
    gm = _sc_gather(h, idx_m)

    sc = params["scorer"]
    w0 = sc[0]["W"]
    out8 = _tc_score(gm, [w0[j * _D:(j + 1) * _D] for j in range(4)],
                     sc[0]["b"].reshape(1, -1),
                     sc[1]["W"], sc[1]["b"].reshape(1, -1),
                     sc[2]["W"].reshape(1, _D), sc[2]["b"].reshape(1, 1),
                     Bb * Mp)
    return out8[:, 0].reshape(Bb, Mp)[:, :Mm]


def kernel(node_features, edge_index, move_nodes, move_mask, params):
    Bb = node_features.shape[0]
    nsplit = 4
    hb = Bb // nsplit
    scores = jnp.concatenate([
        _forward(node_features[i * hb:(i + 1) * hb],
                 edge_index[i * hb:(i + 1) * hb],
                 move_nodes[i * hb:(i + 1) * hb], params)
        for i in range(nsplit)
    ], axis=0)
    return jnp.where(move_mask, scores, -jnp.inf)


# speedup vs baseline: 1.1737x; 1.1737x over previous
import functools

import jax
import jax.numpy as jnp
from jax import lax
from jax.experimental import pallas as pl
from jax.experimental.pallas import tpu as pltpu
from jax.experimental.pallas import tpu_sc as plsc

F32 = jnp.float32
_D = 64
_W = 128
_CH = 128
_NSC = 16


def _silu(x):
    return x * jax.nn.sigmoid(x)


def _vmesh():
    return plsc.VectorSubcoreMesh(core_axis_name="core", subcore_axis_name="subcore")


def _sc_gather(table, idx):
    K = idx.shape[0]

    @functools.partial(
        pl.kernel,
        out_type=jax.ShapeDtypeStruct((K, _W), F32),
        mesh=_vmesh())
    def k(x_hbm, i_hbm, o_hbm):
        def body(i_vmem, o_vmem):
            pltpu.sync_copy(x_hbm.at[i_vmem.at[0]], o_vmem)

        pltpu.emit_pipeline(
            body,
            grid=(K // _CH,),
            in_specs=[pl.BlockSpec((1, _CH), lambda i: (0, i))],
            out_specs=[pl.BlockSpec((_CH, _W), lambda i: (i, 0))],
            core_axis_name=("core", "subcore"),
            dimension_semantics=(pltpu.PARALLEL,),
        )(i_hbm, o_hbm)

    return k(table, idx.reshape(1, K))


_NPASS = 2


def _sc_segment_sum(msg, tgt3, n1p):
    Bb, Ep = msg.shape[0], msg.shape[1]
    nch = tgt3.shape[3]
    ept = nch * _CH
    rng = n1p // _NPASS
    rpt = rng // _NSC
    bpc = Bb // 2

    @functools.partial(
        pl.kernel,
        out_type=jax.ShapeDtypeStruct((Bb, n1p, _W), F32),
        mesh=_vmesh(),
        scratch_types=[
            pltpu.VMEM((nch, _CH), jnp.int32),
            pltpu.VMEM((ept, _W), F32),
            pltpu.VMEM((40, _W), F32),
            pltpu.VMEM_SHARED((rng + 8 * _NSC, _W), F32),
        ])
    def k(msg_hbm, tgt_hbm, o_hbm, idx_v, msg_v, zero_v, acc_sh):
        c = lax.axis_index("core")
        s = lax.axis_index("subcore")

        @pl.loop(0, 40)
        def _(r):
            for l in range(_W // 16):
                zero_v[r, pl.ds(l * 16, 16)] = jnp.zeros((16,), F32)

        for bi in range(bpc):
            b = c * bpc + bi
            pltpu.sync_copy(msg_hbm.at[b, pl.ds(s * ept, ept)], msg_v)
            for p in range(_NPASS):
                for q in range(rpt // 40):
                    pltpu.sync_copy(zero_v,
                                    acc_sh.at[pl.ds(s * rpt + q * 40, 40)])
                plsc.subcore_barrier()
                pltpu.sync_copy(tgt_hbm.at[p, b, s], idx_v)
                for j in range(nch):
                    pltpu.sync_copy(msg_v.at[pl.ds(j * _CH, _CH)],
                                    acc_sh.at[idx_v.at[j]], add=True)
                plsc.subcore_barrier()
                pltpu.sync_copy(acc_sh.at[pl.ds(s * rpt, rpt)],
                                o_hbm.at[b, pl.ds(p * rng + s * rpt, rpt)])

    return k(msg, tgt3)


def _tc_embed(nf, w1, b1, w2, b2):
    R, fin = nf.shape
    RB = R // 32

    def body(nf_ref, w1_ref, b1_ref, w2_ref, b2_ref, o_ref):
        x = jnp.dot(nf_ref[...], w1_ref[...], preferred_element_type=F32) + b1_ref[...]
        x = _silu(x)
        y = jnp.dot(x, w2_ref[...], preferred_element_type=F32) + b2_ref[...]
        o_ref[...] = jnp.concatenate([y, y], axis=1)

    return pl.pallas_call(
        body,
        grid=(R // RB,),
        in_specs=[
            pl.BlockSpec((RB, fin), lambda i: (i, 0)),
            pl.BlockSpec((fin, _D), lambda i: (0, 0)),
            pl.BlockSpec((1, _D), lambda i: (0, 0)),
            pl.BlockSpec((_D, _D), lambda i: (0, 0)),
            pl.BlockSpec((1, _D), lambda i: (0, 0)),
        ],
        out_specs=pl.BlockSpec((RB, _W), lambda i: (i, 0)),
        out_shape=jax.ShapeDtypeStruct((R, _W), F32),
    )(nf, w1, b1, w2, b2)


def _tc_msg(g, w1t, w1s, b1, w2, b2, rows):
    RB = 2048
    nb = rows // RB

    def body(t_ref, s_ref, w1t_ref, w1s_ref, b1_ref, w2_ref, b2_ref, o_ref):
        x = (jnp.dot(t_ref[...][:, :_D], w1t_ref[...],
                     preferred_element_type=F32)
             + jnp.dot(s_ref[...][:, :_D], w1s_ref[...],
                       preferred_element_type=F32)
             + b1_ref[...])
        x = _silu(x)
        m = jnp.dot(x, w2_ref[...], preferred_element_type=F32) + b2_ref[...]
        o_ref[...] = jnp.concatenate([m, jnp.zeros_like(m)], axis=1)

    wspec = lambda shp: pl.BlockSpec(shp, lambda i: (0, 0))
    return pl.pallas_call(
        body,
        grid=(nb,),
        in_specs=[
            pl.BlockSpec((RB, _W), lambda i: (i, 0)),
            pl.BlockSpec((RB, _W), lambda i: (i + nb, 0)),
            wspec((_D, _D)), wspec((_D, _D)), wspec((1, _D)),
            wspec((_D, _D)), wspec((1, _D)),
        ],
        out_specs=pl.BlockSpec((RB, _W), lambda i: (i, 0)),
        out_shape=jax.ShapeDtypeStruct((rows, _W), F32),
    )(g, g, w1t, w1s, b1, w2, b2)


def _tc_upd(h, agg, w1h, w1a, b1, w2, b2, gamma, beta):
    R = h.shape[0]
    RB = R // 32

    def body(h_ref, a_ref, w1h_ref, w1a_ref, b1_ref, w2_ref, b2_ref,
             g_ref, be_ref, o_ref):
        hh = h_ref[...][:, :_D]
        x = (jnp.dot(hh, w1h_ref[...], preferred_element_type=F32)
             + jnp.dot(a_ref[...][:, :_D], w1a_ref[...],
                       preferred_element_type=F32)
             + b1_ref[...])
        x = _silu(x)
        hn = jnp.dot(x, w2_ref[...], preferred_element_type=F32) + b2_ref[...]
        y = hh + hn
        mu = jnp.mean(y, axis=-1, keepdims=True)
        var = jnp.mean((y - mu) ** 2, axis=-1, keepdims=True)
        y = (y - mu) * lax.rsqrt(var + 1e-5) * g_ref[...] + be_ref[...]
        o_ref[...] = jnp.concatenate([y, y], axis=1)

    wspec = lambda shp: pl.BlockSpec(shp, lambda i: (0, 0))
    return pl.pallas_call(
        body,
        grid=(R // RB,),
        in_specs=[
            pl.BlockSpec((RB, _W), lambda i: (i, 0)),
            pl.BlockSpec((RB, _W), lambda i: (i, 0)),
            wspec((_D, _D)), wspec((_D, _D)), wspec((1, _D)),
            wspec((_D, _D)), wspec((1, _D)), wspec((1, _D)), wspec((1, _D)),
        ],
        out_specs=pl.BlockSpec((RB, _W), lambda i: (i, 0)),
        out_shape=jax.ShapeDtypeStruct((R, _W), F32),
    )(h, agg, w1h, w1a, b1, w2, b2, gamma, beta)


def _tc_score(g, w0s, b0, w1, b1, w2row, b2, rows):
    RB = 2048
    nb = rows // RB

    def body(g0, g1, g2, g3, w00, w01, w02, w03, b0_ref, w1_ref, b1_ref,
             w2_ref, b2_ref, o_ref):
        x = (jnp.dot(g0[...][:, :_D], w00[...], preferred_element_type=F32)
             + jnp.dot(g1[...][:, :_D], w01[...], preferred_element_type=F32)
             + jnp.dot(g2[...][:, :_D], w02[...], preferred_element_type=F32)
             + jnp.dot(g3[...][:, :_D], w03[...], preferred_element_type=F32)
             + b0_ref[...])
        x = _silu(x)
        x = _silu(jnp.dot(x, w1_ref[...], preferred_element_type=F32) + b1_ref[...])
        s = jnp.sum(x * w2_ref[...], axis=1, keepdims=True) + b2_ref[...]
        o_ref[...] = jnp.broadcast_to(s, (RB, 8))

    wspec = lambda shp: pl.BlockSpec(shp, lambda i: (0, 0))
    gspec = lambda j: pl.BlockSpec((RB, _W), lambda i, j=j: (i + j * nb, 0))
    return pl.pallas_call(
        body,
        grid=(nb,),
        in_specs=[
            gspec(0), gspec(1), gspec(2), gspec(3),
            wspec((_D, _D)), wspec((_D, _D)), wspec((_D, _D)), wspec((_D, _D)),
            wspec((1, _D)), wspec((_D, _D)), wspec((1, _D)),
            wspec((1, _D)), wspec((1, 1)),
        ],
        out_specs=pl.BlockSpec((RB, 8), lambda i: (i, 0)),
        out_shape=jax.ShapeDtypeStruct((rows, 8), F32),
    )(g, g, g, g, *w0s, b0, w1, b1, w2row, b2)


def _forward(node_features, edge_index, move_nodes, params):
    Bb, n1, fin = node_features.shape
    Ee = edge_index.shape[1]
    Mm = move_nodes.shape[1]
    nalign = _NPASS * _NSC * 8
    n1p = ((n1 + nalign - 1) // nalign) * nalign
    R = Bb * n1p
    grp = _NSC * _CH

    nf = jnp.pad(node_features, ((0, 0), (0, n1p - n1), (0, 0))).reshape(R, fin)
    ne = params["node_embed"]
    h = _tc_embed(nf, ne[0]["W"], ne[0]["b"].reshape(1, -1),
                  ne[1]["W"], ne[1]["b"].reshape(1, -1))

    offs = (jnp.arange(Bb, dtype=jnp.int32) * n1p)[:, None]
    Ep = ((Ee + grp - 1) // grp) * grp
    epad = jnp.full((Bb, Ep - Ee), n1, jnp.int32)
    tgt = jnp.concatenate([edge_index[:, :, 1], epad], axis=1)
    src = jnp.concatenate([edge_index[:, :, 0], epad], axis=1)
    idx_e = jnp.concatenate([(tgt + offs).reshape(-1), (src + offs).reshape(-1)])

    rng = n1p // _NPASS
    ept = Ep // _NSC
    dump = rng + 8 * (jnp.arange(Ep, dtype=jnp.int32) // ept)[None, :]
    tgt3 = []
    for p in range(_NPASS):
        v = tgt - p * rng
        tgt3.append(jnp.where((v >= 0) & (v < rng), v, dump))
    tgt3 = jnp.stack(tgt3).reshape(_NPASS, Bb, _NSC, Ep // grp, _CH)

    for lp in params["layers"]:
        g = _sc_gather(h, idx_e)
        w1 = lp["msg"][0]["W"]
        msg = _tc_msg(g, w1[:_D], w1[_D:], lp["msg"][0]["b"].reshape(1, -1),
                      lp["msg"][1]["W"], lp["msg"][1]["b"].reshape(1, -1),
                      Bb * Ep)
        agg = _sc_segment_sum(msg.reshape(Bb, Ep, _W), tgt3, n1p)
        wu = lp["upd"][0]["W"]
        h = _tc_upd(h, agg.reshape(R, _W), wu[:_D], wu[_D:],
                    lp["upd"][0]["b"].reshape(1, -1), lp["upd"][1]["W"],
                    lp["upd"][1]["b"].reshape(1, -1),
                    lp["gamma"].reshape(1, -1), lp["beta"].reshape(1, -1))

    Mp = ((Mm + _CH - 1) // _CH) * _CH
    mvi = jnp.concatenate(
        [move_nodes.transpose(0, 2, 1),
         jnp.full((Bb, 4, Mp - Mm), n1, jnp.int32)], axis=2)
    idx_m = (mvi + offs[:, :, None]).transpose(1, 0, 2).reshape(-1)
    gm = _sc_gather(h, idx_m)

    sc = params["scorer"]
    w0 = sc[0]["W"]
    out8 = _tc_score(gm, [w0[j * _D:(j + 1) * _D] for j in range(4)],
                     sc[0]["b"].reshape(1, -1),
                     sc[1]["W"], sc[1]["b"].reshape(1, -1),
                     sc[2]["W"].reshape(1, _D), sc[2]["b"].reshape(1, 1),
                     Bb * Mp)
    return out8[:, 0].reshape(Bb, Mp)[:, :Mm]


def kernel(node_features, edge_index, move_nodes, move_mask, params):
    Bb = node_features.shape[0]
    nsplit = 2
    hb = Bb // nsplit
    scores = jnp.concatenate([
        _forward(node_features[i * hb:(i + 1) * hb],
                 edge_index[i * hb:(i + 1) * hb],
                 move_nodes[i * hb:(i + 1) * hb], params)
        for i in range(nsplit)
    ], axis=0)
    return jnp.where(move_mask, scores, -jnp.inf)


# speedup vs baseline: 1.1761x; 1.0020x over previous
import functools

import jax
import jax.numpy as jnp
from jax import lax
from jax.experimental import pallas as pl
from jax.experimental.pallas import tpu as pltpu
from jax.experimental.pallas import tpu_sc as plsc

F32 = jnp.float32
_D = 64
_W = 128
_CH = 128
_NSC = 16


def _silu(x):
    return x * jax.nn.sigmoid(x)


def _vmesh():
    return plsc.VectorSubcoreMesh(core_axis_name="core", subcore_axis_name="subcore")


def _sc_gather(table, idx):
    K = idx.shape[0]
    cpw = K // (2 * _NSC) // _CH

    @functools.partial(
        pl.kernel,
        out_type=jax.ShapeDtypeStruct((K, _W), F32),
        mesh=_vmesh(),
        scratch_types=[
            pltpu.VMEM((cpw, _CH), jnp.int32),
            pltpu.VMEM((_CH, _W), F32),
            pltpu.VMEM((_CH, _W), F32),
            pltpu.SemaphoreType.DMA,
            pltpu.SemaphoreType.DMA,
        ])
    def k(x_hbm, i_hbm, o_hbm, idx_v, buf_a, buf_b, sem_a, sem_b):
        c = lax.axis_index("core")
        s = lax.axis_index("subcore")
        w = s * 2 + c
        pltpu.sync_copy(i_hbm.at[w], idx_v)
        base = w * cpw * _CH
        bufs = (buf_a, buf_b)
        sems = (sem_a, sem_b)
        pltpu.async_copy(x_hbm.at[idx_v.at[0]], buf_a, sem_a)
        for ch in range(cpw):
            sl = ch % 2
            pltpu.make_async_copy(x_hbm.at[idx_v.at[ch]], bufs[sl],
                                  sems[sl]).wait()
            if ch + 1 < cpw:
                pltpu.async_copy(x_hbm.at[idx_v.at[ch + 1]],
                                 bufs[1 - sl], sems[1 - sl])
            pltpu.sync_copy(bufs[sl], o_hbm.at[pl.ds(base + ch * _CH, _CH)])

    return k(table, idx.reshape(2 * _NSC, cpw, _CH))


_NPASS = 2


def _sc_segment_sum(msg, tgt3, n1p):
    Bb, Ep = msg.shape[0], msg.shape[1]
    nch = tgt3.shape[3]
    ept = nch * _CH
    rng = n1p // _NPASS
    rpt = rng // _NSC
    bpc = Bb // 2

    @functools.partial(
        pl.kernel,
        out_type=jax.ShapeDtypeStruct((Bb, n1p, _W), F32),
        mesh=_vmesh(),
        scratch_types=[
            pltpu.VMEM((nch, _CH), jnp.int32),
            pltpu.VMEM((ept, _W), F32),
            pltpu.VMEM((40, _W), F32),
            pltpu.VMEM_SHARED((rng + 8 * _NSC, _W), F32),
        ])
    def k(msg_hbm, tgt_hbm, o_hbm, idx_v, msg_v, zero_v, acc_sh):
        c = lax.axis_index("core")
        s = lax.axis_index("subcore")

        @pl.loop(0, 40)
        def _(r):
            for l in range(_W // 16):
                zero_v[r, pl.ds(l * 16, 16)] = jnp.zeros((16,), F32)

        for bi in range(bpc):
            b = c * bpc + bi
            pltpu.sync_copy(msg_hbm.at[b, pl.ds(s * ept, ept)], msg_v)
            for p in range(_NPASS):
                for q in range(rpt // 40):
                    pltpu.sync_copy(zero_v,
                                    acc_sh.at[pl.ds(s * rpt + q * 40, 40)])
                plsc.subcore_barrier()
                pltpu.sync_copy(tgt_hbm.at[p, b, s], idx_v)
                for j in range(nch):
                    pltpu.sync_copy(msg_v.at[pl.ds(j * _CH, _CH)],
                                    acc_sh.at[idx_v.at[j]], add=True)
                plsc.subcore_barrier()
                pltpu.sync_copy(acc_sh.at[pl.ds(s * rpt, rpt)],
                                o_hbm.at[b, pl.ds(p * rng + s * rpt, rpt)])

    return k(msg, tgt3)


def _tc_embed(nf, w1, b1, w2, b2):
    R, fin = nf.shape
    RB = R // 32

    def body(nf_ref, w1_ref, b1_ref, w2_ref, b2_ref, o_ref):
        x = jnp.dot(nf_ref[...], w1_ref[...], preferred_element_type=F32) + b1_ref[...]
        x = _silu(x)
        y = jnp.dot(x, w2_ref[...], preferred_element_type=F32) + b2_ref[...]
        o_ref[...] = jnp.concatenate([y, y], axis=1)

    return pl.pallas_call(
        body,
        grid=(R // RB,),
        in_specs=[
            pl.BlockSpec((RB, fin), lambda i: (i, 0)),
            pl.BlockSpec((fin, _D), lambda i: (0, 0)),
            pl.BlockSpec((1, _D), lambda i: (0, 0)),
            pl.BlockSpec((_D, _D), lambda i: (0, 0)),
            pl.BlockSpec((1, _D), lambda i: (0, 0)),
        ],
        out_specs=pl.BlockSpec((RB, _W), lambda i: (i, 0)),
        out_shape=jax.ShapeDtypeStruct((R, _W), F32),
    )(nf, w1, b1, w2, b2)


def _tc_msg(g, w1t, w1s, b1, w2, b2, rows):
    RB = 2048
    nb = rows // RB

    def body(t_ref, s_ref, w1t_ref, w1s_ref, b1_ref, w2_ref, b2_ref, o_ref):
        x = (jnp.dot(t_ref[...][:, :_D], w1t_ref[...],
                     preferred_element_type=F32)
             + jnp.dot(s_ref[...][:, :_D], w1s_ref[...],
                       preferred_element_type=F32)
             + b1_ref[...])
        x = _silu(x)
        m = jnp.dot(x, w2_ref[...], preferred_element_type=F32) + b2_ref[...]
        o_ref[...] = jnp.concatenate([m, jnp.zeros_like(m)], axis=1)

    wspec = lambda shp: pl.BlockSpec(shp, lambda i: (0, 0))
    return pl.pallas_call(
        body,
        grid=(nb,),
        in_specs=[
            pl.BlockSpec((RB, _W), lambda i: (i, 0)),
            pl.BlockSpec((RB, _W), lambda i: (i + nb, 0)),
            wspec((_D, _D)), wspec((_D, _D)), wspec((1, _D)),
            wspec((_D, _D)), wspec((1, _D)),
        ],
        out_specs=pl.BlockSpec((RB, _W), lambda i: (i, 0)),
        out_shape=jax.ShapeDtypeStruct((rows, _W), F32),
    )(g, g, w1t, w1s, b1, w2, b2)


def _tc_upd(h, agg, w1h, w1a, b1, w2, b2, gamma, beta):
    R = h.shape[0]
    RB = R // 32

    def body(h_ref, a_ref, w1h_ref, w1a_ref, b1_ref, w2_ref, b2_ref,
             g_ref, be_ref, o_ref):
        hh = h_ref[...][:, :_D]
        x = (jnp.dot(hh, w1h_ref[...], preferred_element_type=F32)
             + jnp.dot(a_ref[...][:, :_D], w1a_ref[...],
                       preferred_element_type=F32)
             + b1_ref[...])
        x = _silu(x)
        hn = jnp.dot(x, w2_ref[...], preferred_element_type=F32) + b2_ref[...]
        y = hh + hn
        mu = jnp.mean(y, axis=-1, keepdims=True)
        var = jnp.mean((y - mu) ** 2, axis=-1, keepdims=True)
        y = (y - mu) * lax.rsqrt(var + 1e-5) * g_ref[...] + be_ref[...]
        o_ref[...] = jnp.concatenate([y, y], axis=1)

    wspec = lambda shp: pl.BlockSpec(shp, lambda i: (0, 0))
    return pl.pallas_call(
        body,
        grid=(R // RB,),
        in_specs=[
            pl.BlockSpec((RB, _W), lambda i: (i, 0)),
            pl.BlockSpec((RB, _W), lambda i: (i, 0)),
            wspec((_D, _D)), wspec((_D, _D)), wspec((1, _D)),
            wspec((_D, _D)), wspec((1, _D)), wspec((1, _D)), wspec((1, _D)),
        ],
        out_specs=pl.BlockSpec((RB, _W), lambda i: (i, 0)),
        out_shape=jax.ShapeDtypeStruct((R, _W), F32),
    )(h, agg, w1h, w1a, b1, w2, b2, gamma, beta)


def _tc_score(g, w0s, b0, w1, b1, w2row, b2, rows):
    RB = 2048
    nb = rows // RB

    def body(g0, g1, g2, g3, w00, w01, w02, w03, b0_ref, w1_ref, b1_ref,
             w2_ref, b2_ref, o_ref):
        x = (jnp.dot(g0[...][:, :_D], w00[...], preferred_element_type=F32)
             + jnp.dot(g1[...][:, :_D], w01[...], preferred_element_type=F32)
             + jnp.dot(g2[...][:, :_D], w02[...], preferred_element_type=F32)
             + jnp.dot(g3[...][:, :_D], w03[...], preferred_element_type=F32)
             + b0_ref[...])
        x = _silu(x)
        x = _silu(jnp.dot(x, w1_ref[...], preferred_element_type=F32) + b1_ref[...])
        s = jnp.sum(x * w2_ref[...], axis=1, keepdims=True) + b2_ref[...]
        o_ref[...] = jnp.broadcast_to(s, (RB, 8))

    wspec = lambda shp: pl.BlockSpec(shp, lambda i: (0, 0))
    gspec = lambda j: pl.BlockSpec((RB, _W), lambda i, j=j: (i + j * nb, 0))
    return pl.pallas_call(
        body,
        grid=(nb,),
        in_specs=[
            gspec(0), gspec(1), gspec(2), gspec(3),
            wspec((_D, _D)), wspec((_D, _D)), wspec((_D, _D)), wspec((_D, _D)),
            wspec((1, _D)), wspec((_D, _D)), wspec((1, _D)),
            wspec((1, _D)), wspec((1, 1)),
        ],
        out_specs=pl.BlockSpec((RB, 8), lambda i: (i, 0)),
        out_shape=jax.ShapeDtypeStruct((rows, 8), F32),
    )(g, g, g, g, *w0s, b0, w1, b1, w2row, b2)


def _forward(node_features, edge_index, move_nodes, params):
    Bb, n1, fin = node_features.shape
    Ee = edge_index.shape[1]
    Mm = move_nodes.shape[1]
    nalign = _NPASS * _NSC * 8
    n1p = ((n1 + nalign - 1) // nalign) * nalign
    R = Bb * n1p
    grp = _NSC * _CH

    nf = jnp.pad(node_features, ((0, 0), (0, n1p - n1), (0, 0))).reshape(R, fin)
    ne = params["node_embed"]
    h = _tc_embed(nf, ne[0]["W"], ne[0]["b"].reshape(1, -1),
                  ne[1]["W"], ne[1]["b"].reshape(1, -1))

    offs = (jnp.arange(Bb, dtype=jnp.int32) * n1p)[:, None]
    Ep = ((Ee + grp - 1) // grp) * grp
    epad = jnp.full((Bb, Ep - Ee), n1, jnp.int32)
    tgt = jnp.concatenate([edge_index[:, :, 1], epad], axis=1)
    src = jnp.concatenate([edge_index[:, :, 0], epad], axis=1)
    idx_e = jnp.concatenate([(tgt + offs).reshape(-1), (src + offs).reshape(-1)])

    rng = n1p // _NPASS
    ept = Ep // _NSC
    dump = rng + 8 * (jnp.arange(Ep, dtype=jnp.int32) // ept)[None, :]
    tgt3 = []
    for p in range(_NPASS):
        v = tgt - p * rng
        tgt3.append(jnp.where((v >= 0) & (v < rng), v, dump))
    tgt3 = jnp.stack(tgt3).reshape(_NPASS, Bb, _NSC, Ep // grp, _CH)

    for lp in params["layers"]:
        g = _sc_gather(h, idx_e)
        w1 = lp["msg"][0]["W"]
        msg = _tc_msg(g, w1[:_D], w1[_D:], lp["msg"][0]["b"].reshape(1, -1),
                      lp["msg"][1]["W"], lp["msg"][1]["b"].reshape(1, -1),
                      Bb * Ep)
        agg = _sc_segment_sum(msg.reshape(Bb, Ep, _W), tgt3, n1p)
        wu = lp["upd"][0]["W"]
        h = _tc_upd(h, agg.reshape(R, _W), wu[:_D], wu[_D:],
                    lp["upd"][0]["b"].reshape(1, -1), lp["upd"][1]["W"],
                    lp["upd"][1]["b"].reshape(1, -1),
                    lp["gamma"].reshape(1, -1), lp["beta"].reshape(1, -1))

    Mp = ((Mm + _CH - 1) // _CH) * _CH
    mvi = jnp.concatenate(
        [move_nodes.transpose(0, 2, 1),
         jnp.full((Bb, 4, Mp - Mm), n1, jnp.int32)], axis=2)
    idx_m = (mvi + offs[:, :, None]).transpose(1, 0, 2).reshape(-1)
    gm = _sc_gather(h, idx_m)

    sc = params["scorer"]
    w0 = sc[0]["W"]
    out8 = _tc_score(gm, [w0[j * _D:(j + 1) * _D] for j in range(4)],
                     sc[0]["b"].reshape(1, -1),
                     sc[1]["W"], sc[1]["b"].reshape(1, -1),
                     sc[2]["W"].reshape(1, _D), sc[2]["b"].reshape(1, 1),
                     Bb * Mp)
    return out8[:, 0].reshape(Bb, Mp)[:, :Mm]


def kernel(node_features, edge_index, move_nodes, move_mask, params):
    Bb = node_features.shape[0]
    nsplit = 2
    hb = Bb // nsplit
    scores = jnp.concatenate([
        _forward(node_features[i * hb:(i + 1) * hb],
                 edge_index[i * hb:(i + 1) * hb],
                 move_nodes[i * hb:(i + 1) * hb], params)
        for i in range(nsplit)
    ], axis=0)
    return jnp.where(move_mask, scores, -jnp.inf)


# speedup vs baseline: 1.1882x; 1.0103x over previous
import functools

import jax
import jax.numpy as jnp
from jax import lax
from jax.experimental import pallas as pl
from jax.experimental.pallas import tpu as pltpu
from jax.experimental.pallas import tpu_sc as plsc

F32 = jnp.float32
_D = 64
_W = 128
_CH = 128
_NSC = 16


def _silu(x):
    return x * jax.nn.sigmoid(x)


def _vmesh():
    return plsc.VectorSubcoreMesh(core_axis_name="core", subcore_axis_name="subcore")


def _sc_gather(table, idx):
    K = idx.shape[0]
    cpw = K // (2 * _NSC) // _CH

    @functools.partial(
        pl.kernel,
        out_type=jax.ShapeDtypeStruct((K, _W), F32),
        mesh=_vmesh(),
        scratch_types=[
            pltpu.VMEM((cpw, _CH), jnp.int32),
            pltpu.VMEM((_CH, _W), F32),
            pltpu.VMEM((_CH, _W), F32),
            pltpu.SemaphoreType.DMA,
            pltpu.SemaphoreType.DMA,
        ])
    def k(x_hbm, i_hbm, o_hbm, idx_v, buf_a, buf_b, sem_a, sem_b):
        c = lax.axis_index("core")
        s = lax.axis_index("subcore")
        w = s * 2 + c
        pltpu.sync_copy(i_hbm.at[w], idx_v)
        base = w * cpw * _CH
        bufs = (buf_a, buf_b)
        sems = (sem_a, sem_b)
        pltpu.async_copy(x_hbm.at[idx_v.at[0]], buf_a, sem_a)
        for ch in range(cpw):
            sl = ch % 2
            pltpu.make_async_copy(x_hbm.at[idx_v.at[ch]], bufs[sl],
                                  sems[sl]).wait()
            if ch + 1 < cpw:
                pltpu.async_copy(x_hbm.at[idx_v.at[ch + 1]],
                                 bufs[1 - sl], sems[1 - sl])
            pltpu.sync_copy(bufs[sl], o_hbm.at[pl.ds(base + ch * _CH, _CH)])

    return k(table, idx.reshape(2 * _NSC, cpw, _CH))


_NPASS = 2


def _sc_segment_sum(msg, tgt3, n1p):
    Bb, Ep = msg.shape[0], msg.shape[1]
    nch = tgt3.shape[3]
    ept = nch * _CH
    rng = n1p // _NPASS
    rpt = rng // _NSC
    bpc = Bb // 2

    @functools.partial(
        pl.kernel,
        out_type=jax.ShapeDtypeStruct((Bb, n1p, _W), F32),
        mesh=_vmesh(),
        scratch_types=[
            pltpu.VMEM((nch, _CH), jnp.int32),
            pltpu.VMEM((ept, _W), F32),
            pltpu.VMEM((40, _W), F32),
            pltpu.VMEM_SHARED((rng + 8 * _NSC, _W), F32),
            pltpu.SemaphoreType.DMA,
        ])
    def k(msg_hbm, tgt_hbm, o_hbm, idx_v, msg_v, zero_v, acc_sh, msem):
        c = lax.axis_index("core")
        s = lax.axis_index("subcore")

        @pl.loop(0, 40)
        def _(r):
            for l in range(_W // 16):
                zero_v[r, pl.ds(l * 16, 16)] = jnp.zeros((16,), F32)

        for bi in range(bpc):
            b = c * bpc + bi
            mcopy = pltpu.async_copy(msg_hbm.at[b, pl.ds(s * ept, ept)],
                                     msg_v, msem)
            for p in range(_NPASS):
                for q in range(rpt // 40):
                    pltpu.sync_copy(zero_v,
                                    acc_sh.at[pl.ds(s * rpt + q * 40, 40)])
                plsc.subcore_barrier()
                pltpu.sync_copy(tgt_hbm.at[p, b, s], idx_v)
                if p == 0:
                    mcopy.wait()
                for j in range(nch):
                    pltpu.sync_copy(msg_v.at[pl.ds(j * _CH, _CH)],
                                    acc_sh.at[idx_v.at[j]], add=True)
                plsc.subcore_barrier()
                pltpu.sync_copy(acc_sh.at[pl.ds(s * rpt, rpt)],
                                o_hbm.at[b, pl.ds(p * rng + s * rpt, rpt)])

    return k(msg, tgt3)


def _tc_embed(nf, w1, b1, w2, b2):
    R, fin = nf.shape
    RB = R // 32

    def body(nf_ref, w1_ref, b1_ref, w2_ref, b2_ref, o_ref):
        x = jnp.dot(nf_ref[...], w1_ref[...], preferred_element_type=F32) + b1_ref[...]
        x = _silu(x)
        y = jnp.dot(x, w2_ref[...], preferred_element_type=F32) + b2_ref[...]
        o_ref[...] = jnp.concatenate([y, y], axis=1)

    return pl.pallas_call(
        body,
        grid=(R // RB,),
        in_specs=[
            pl.BlockSpec((RB, fin), lambda i: (i, 0)),
            pl.BlockSpec((fin, _D), lambda i: (0, 0)),
            pl.BlockSpec((1, _D), lambda i: (0, 0)),
            pl.BlockSpec((_D, _D), lambda i: (0, 0)),
            pl.BlockSpec((1, _D), lambda i: (0, 0)),
        ],
        out_specs=pl.BlockSpec((RB, _W), lambda i: (i, 0)),
        out_shape=jax.ShapeDtypeStruct((R, _W), F32),
    )(nf, w1, b1, w2, b2)


def _tc_msg(g, w1t, w1s, b1, w2, b2, rows):
    RB = 2048
    nb = rows // RB

    def body(t_ref, s_ref, w1t_ref, w1s_ref, b1_ref, w2_ref, b2_ref, o_ref):
        x = (jnp.dot(t_ref[...][:, :_D], w1t_ref[...],
                     preferred_element_type=F32)
             + jnp.dot(s_ref[...][:, :_D], w1s_ref[...],
                       preferred_element_type=F32)
             + b1_ref[...])
        x = _silu(x)
        m = jnp.dot(x, w2_ref[...], preferred_element_type=F32) + b2_ref[...]
        o_ref[...] = jnp.concatenate([m, jnp.zeros_like(m)], axis=1)

    wspec = lambda shp: pl.BlockSpec(shp, lambda i: (0, 0))
    return pl.pallas_call(
        body,
        grid=(nb,),
        in_specs=[
            pl.BlockSpec((RB, _W), lambda i: (i, 0)),
            pl.BlockSpec((RB, _W), lambda i: (i + nb, 0)),
            wspec((_D, _D)), wspec((_D, _D)), wspec((1, _D)),
            wspec((_D, _D)), wspec((1, _D)),
        ],
        out_specs=pl.BlockSpec((RB, _W), lambda i: (i, 0)),
        out_shape=jax.ShapeDtypeStruct((rows, _W), F32),
    )(g, g, w1t, w1s, b1, w2, b2)


def _tc_upd(h, agg, w1h, w1a, b1, w2, b2, gamma, beta):
    R = h.shape[0]
    RB = R // 32

    def body(h_ref, a_ref, w1h_ref, w1a_ref, b1_ref, w2_ref, b2_ref,
             g_ref, be_ref, o_ref):
        hh = h_ref[...][:, :_D]
        x = (jnp.dot(hh, w1h_ref[...], preferred_element_type=F32)
             + jnp.dot(a_ref[...][:, :_D], w1a_ref[...],
                       preferred_element_type=F32)
             + b1_ref[...])
        x = _silu(x)
        hn = jnp.dot(x, w2_ref[...], preferred_element_type=F32) + b2_ref[...]
        y = hh + hn
        mu = jnp.mean(y, axis=-1, keepdims=True)
        var = jnp.mean((y - mu) ** 2, axis=-1, keepdims=True)
        y = (y - mu) * lax.rsqrt(var + 1e-5) * g_ref[...] + be_ref[...]
        o_ref[...] = jnp.concatenate([y, y], axis=1)

    wspec = lambda shp: pl.BlockSpec(shp, lambda i: (0, 0))
    return pl.pallas_call(
        body,
        grid=(R // RB,),
        in_specs=[
            pl.BlockSpec((RB, _W), lambda i: (i, 0)),
            pl.BlockSpec((RB, _W), lambda i: (i, 0)),
            wspec((_D, _D)), wspec((_D, _D)), wspec((1, _D)),
            wspec((_D, _D)), wspec((1, _D)), wspec((1, _D)), wspec((1, _D)),
        ],
        out_specs=pl.BlockSpec((RB, _W), lambda i: (i, 0)),
        out_shape=jax.ShapeDtypeStruct((R, _W), F32),
    )(h, agg, w1h, w1a, b1, w2, b2, gamma, beta)


def _tc_score(g, w0s, b0, w1, b1, w2row, b2, rows):
    RB = 2048
    nb = rows // RB

    def body(g0, g1, g2, g3, w00, w01, w02, w03, b0_ref, w1_ref, b1_ref,
             w2_ref, b2_ref, o_ref):
        x = (jnp.dot(g0[...][:, :_D], w00[...], preferred_element_type=F32)
             + jnp.dot(g1[...][:, :_D], w01[...], preferred_element_type=F32)
             + jnp.dot(g2[...][:, :_D], w02[...], preferred_element_type=F32)
             + jnp.dot(g3[...][:, :_D], w03[...], preferred_element_type=F32)
             + b0_ref[...])
        x = _silu(x)
        x = _silu(jnp.dot(x, w1_ref[...], preferred_element_type=F32) + b1_ref[...])
        s = jnp.sum(x * w2_ref[...], axis=1, keepdims=True) + b2_ref[...]
        o_ref[...] = jnp.broadcast_to(s, (RB, 8))

    wspec = lambda shp: pl.BlockSpec(shp, lambda i: (0, 0))
    gspec = lambda j: pl.BlockSpec((RB, _W), lambda i, j=j: (i + j * nb, 0))
    return pl.pallas_call(
        body,
        grid=(nb,),
        in_specs=[
            gspec(0), gspec(1), gspec(2), gspec(3),
            wspec((_D, _D)), wspec((_D, _D)), wspec((_D, _D)), wspec((_D, _D)),
            wspec((1, _D)), wspec((_D, _D)), wspec((1, _D)),
            wspec((1, _D)), wspec((1, 1)),
        ],
        out_specs=pl.BlockSpec((RB, 8), lambda i: (i, 0)),
        out_shape=jax.ShapeDtypeStruct((rows, 8), F32),
    )(g, g, g, g, *w0s, b0, w1, b1, w2row, b2)


def _forward(node_features, edge_index, move_nodes, params):
    Bb, n1, fin = node_features.shape
    Ee = edge_index.shape[1]
    Mm = move_nodes.shape[1]
    nalign = _NPASS * _NSC * 8
    n1p = ((n1 + nalign - 1) // nalign) * nalign
    R = Bb * n1p
    grp = _NSC * _CH

    nf = jnp.pad(node_features, ((0, 0), (0, n1p - n1), (0, 0))).reshape(R, fin)
    ne = params["node_embed"]
    h = _tc_embed(nf, ne[0]["W"], ne[0]["b"].reshape(1, -1),
                  ne[1]["W"], ne[1]["b"].reshape(1, -1))

    offs = (jnp.arange(Bb, dtype=jnp.int32) * n1p)[:, None]
    Ep = ((Ee + grp - 1) // grp) * grp
    epad = jnp.full((Bb, Ep - Ee), n1, jnp.int32)
    tgt = jnp.concatenate([edge_index[:, :, 1], epad], axis=1)
    src = jnp.concatenate([edge_index[:, :, 0], epad], axis=1)
    idx_e = jnp.concatenate([(tgt + offs).reshape(-1), (src + offs).reshape(-1)])

    rng = n1p // _NPASS
    ept = Ep // _NSC
    dump = rng + 8 * (jnp.arange(Ep, dtype=jnp.int32) // ept)[None, :]
    tgt3 = []
    for p in range(_NPASS):
        v = tgt - p * rng
        tgt3.append(jnp.where((v >= 0) & (v < rng), v, dump))
    tgt3 = jnp.stack(tgt3).reshape(_NPASS, Bb, _NSC, Ep // grp, _CH)

    for lp in params["layers"]:
        g = _sc_gather(h, idx_e)
        w1 = lp["msg"][0]["W"]
        msg = _tc_msg(g, w1[:_D], w1[_D:], lp["msg"][0]["b"].reshape(1, -1),
                      lp["msg"][1]["W"], lp["msg"][1]["b"].reshape(1, -1),
                      Bb * Ep)
        agg = _sc_segment_sum(msg.reshape(Bb, Ep, _W), tgt3, n1p)
        wu = lp["upd"][0]["W"]
        h = _tc_upd(h, agg.reshape(R, _W), wu[:_D], wu[_D:],
                    lp["upd"][0]["b"].reshape(1, -1), lp["upd"][1]["W"],
                    lp["upd"][1]["b"].reshape(1, -1),
                    lp["gamma"].reshape(1, -1), lp["beta"].reshape(1, -1))

    Mp = ((Mm + _CH - 1) // _CH) * _CH
    mvi = jnp.concatenate(
        [move_nodes.transpose(0, 2, 1),
         jnp.full((Bb, 4, Mp - Mm), n1, jnp.int32)], axis=2)
    idx_m = (mvi + offs[:, :, None]).transpose(1, 0, 2).reshape(-1)
    gm = _sc_gather(h, idx_m)

    sc = params["scorer"]
    w0 = sc[0]["W"]
    out8 = _tc_score(gm, [w0[j * _D:(j + 1) * _D] for j in range(4)],
                     sc[0]["b"].reshape(1, -1),
                     sc[1]["W"], sc[1]["b"].reshape(1, -1),
                     sc[2]["W"].reshape(1, _D), sc[2]["b"].reshape(1, 1),
                     Bb * Mp)
    return out8[:, 0].reshape(Bb, Mp)[:, :Mm]


def kernel(node_features, edge_index, move_nodes, move_mask, params):
    Bb = node_features.shape[0]
    nsplit = 2
    hb = Bb // nsplit
    scores = jnp.concatenate([
        _forward(node_features[i * hb:(i + 1) * hb],
                 edge_index[i * hb:(i + 1) * hb],
                 move_nodes[i * hb:(i + 1) * hb], params)
        for i in range(nsplit)
    ], axis=0)
    return jnp.where(move_mask, scores, -jnp.inf)
